# baseline (speedup 1.0000x reference)
import functools

import jax
import jax.numpy as jnp
from jax import lax
from jax.experimental import pallas as pl
from jax.experimental.pallas import tpu as pltpu
from jax.experimental.pallas import tpu_sc as plsc

VOCAB = 100000
TOKEN_DIM = 128
FEAT_DIM = 512
FEAT_EMB_DIM = 128
OUT_DIM = TOKEN_DIM + FEAT_EMB_DIM
B, L = 1024, 50
LP = 56
N_PAD_ROWS = B * LP

NC = 2
NS = 16
NW = NC * NS
BATCH_PER_W = B // NW
ROWS_PER_W = BATCH_PER_W * LP
CHUNK = 128
N_CHUNKS = ROWS_PER_W // CHUNK


def _sc_gather(table, idx_pad):
    mesh = plsc.VectorSubcoreMesh(core_axis_name="c", subcore_axis_name="s")

    @functools.partial(
        pl.kernel,
        mesh=mesh,
        out_type=jax.ShapeDtypeStruct((N_PAD_ROWS, TOKEN_DIM), jnp.float32),
        scratch_types=[
            pltpu.VMEM((ROWS_PER_W,), jnp.int32),
            pltpu.VMEM((CHUNK, TOKEN_DIM), jnp.float32),
            pltpu.VMEM((CHUNK, TOKEN_DIM), jnp.float32),
            pltpu.SemaphoreType.DMA,
            pltpu.SemaphoreType.DMA,
        ],
    )
    def gather_k(table_hbm, idx_hbm, out_hbm, idx_v, rows0, rows1, sem0, sem1):
        wid = lax.axis_index("s") * NC + lax.axis_index("c")
        base = wid * ROWS_PER_W
        pltpu.sync_copy(idx_hbm.at[pl.ds(base, ROWS_PER_W)], idx_v)

        bufs = (rows0, rows1)
        sems = (sem0, sem1)
        copies = []
        for c in range(N_CHUNKS):
            off = c * CHUNK
            buf = bufs[c % 2]
            cp = pltpu.make_async_copy(
                table_hbm.at[idx_v.at[pl.ds(off, CHUNK)]],
                buf,
                sems[c % 2],
            )
            cp.start()
            copies.append((cp, off, buf))
            if c >= 1:
                pcp, poff, pbuf = copies[c - 1]
                pcp.wait()
                pltpu.sync_copy(pbuf, out_hbm.at[pl.ds(base + poff, CHUNK)])
        lcp, loff, lbuf = copies[-1]
        lcp.wait()
        pltpu.sync_copy(lbuf, out_hbm.at[pl.ds(base + loff, CHUNK)])

    return gather_k(table, idx_pad)


BB = 128


def _tc_body(g_ref, f_ref, w_ref, b_ref, o_ref):
    o_ref[:, :, :TOKEN_DIM] = g_ref[:, :L, :]
    acc = lax.dot_general(
        f_ref[...],
        w_ref[...],
        (((2,), (1,)), ((), ())),
        preferred_element_type=jnp.float32,
    )
    o_ref[:, :, TOKEN_DIM:] = acc + b_ref[...]


def _tc_project_concat(gathered3d, features, W, b3d):
    return pl.pallas_call(
        _tc_body,
        grid=(B // BB,),
        in_specs=[
            pl.BlockSpec((BB, LP, TOKEN_DIM), lambda i: (i, 0, 0)),
            pl.BlockSpec((BB, L, FEAT_DIM), lambda i: (i, 0, 0)),
            pl.BlockSpec((FEAT_EMB_DIM, FEAT_DIM), lambda i: (0, 0)),
            pl.BlockSpec((1, 1, FEAT_EMB_DIM), lambda i: (0, 0, 0)),
        ],
        out_specs=pl.BlockSpec((BB, L, OUT_DIM), lambda i: (i, 0, 0)),
        out_shape=jax.ShapeDtypeStruct((B, L, OUT_DIM), jnp.float32),
    )(gathered3d, features, W, b3d)


@jax.jit
def kernel(tokens, features, table, W, b):
    pad_idx = (
        jnp.arange(B, dtype=jnp.int32)[:, None] * (LP - L)
        + jnp.arange(LP - L, dtype=jnp.int32)[None, :]
    )
    idx_pad = jnp.concatenate([tokens.astype(jnp.int32), pad_idx], axis=1)
    gathered = _sc_gather(table, idx_pad.reshape(N_PAD_ROWS))
    out = _tc_project_concat(
        gathered.reshape(B, LP, TOKEN_DIM),
        features,
        W,
        b.reshape(1, 1, FEAT_EMB_DIM),
    )
    return out


# speedup vs baseline: 1.0041x; 1.0041x over previous
import functools

import jax
import jax.numpy as jnp
from jax import lax
from jax.experimental import pallas as pl
from jax.experimental.pallas import tpu as pltpu
from jax.experimental.pallas import tpu_sc as plsc

VOCAB = 100000
TOKEN_DIM = 128
FEAT_DIM = 512
FEAT_EMB_DIM = 128
OUT_DIM = TOKEN_DIM + FEAT_EMB_DIM
B, L = 1024, 50
LP = 56
N_PAD_ROWS = B * LP

NC = 2
NS = 16
NW = NC * NS
BATCH_PER_W = B // NW
ROWS_PER_W = BATCH_PER_W * LP
CHUNK = 128
N_CHUNKS = ROWS_PER_W // CHUNK
NBUF = 4


def _sc_gather(table, idx_pad):
    mesh = plsc.VectorSubcoreMesh(core_axis_name="c", subcore_axis_name="s")

    @functools.partial(
        pl.kernel,
        mesh=mesh,
        out_type=jax.ShapeDtypeStruct((N_PAD_ROWS, TOKEN_DIM), jnp.float32),
        scratch_types=[
            pltpu.VMEM((ROWS_PER_W,), jnp.int32),
            pltpu.VMEM((NBUF, CHUNK, TOKEN_DIM), jnp.float32),
            [pltpu.SemaphoreType.DMA] * NBUF,
            [pltpu.SemaphoreType.DMA] * NBUF,
        ],
    )
    def gather_k(table_hbm, idx_hbm, out_hbm, idx_v, rows, gsems, wsems):
        wid = lax.axis_index("s") * NC + lax.axis_index("c")
        base = wid * ROWS_PER_W
        pltpu.sync_copy(idx_hbm.at[pl.ds(base, ROWS_PER_W)], idx_v)

        gathers = []
        writes = []
        for c in range(N_CHUNKS):
            off = c * CHUNK
            s = c % NBUF
            if c >= NBUF:
                writes[c - NBUF].wait()
            gcp = pltpu.make_async_copy(
                table_hbm.at[idx_v.at[pl.ds(off, CHUNK)]],
                rows.at[s],
                gsems[s],
            )
            gcp.start()
            gathers.append(gcp)
            if c >= 1:
                ps = (c - 1) % NBUF
                gathers[c - 1].wait()
                wcp = pltpu.make_async_copy(
                    rows.at[ps],
                    out_hbm.at[pl.ds(base + (c - 1) * CHUNK, CHUNK)],
                    wsems[ps],
                )
                wcp.start()
                writes.append(wcp)
        gathers[-1].wait()
        lcp = pltpu.make_async_copy(
            rows.at[(N_CHUNKS - 1) % NBUF],
            out_hbm.at[pl.ds(base + (N_CHUNKS - 1) * CHUNK, CHUNK)],
            wsems[(N_CHUNKS - 1) % NBUF],
        )
        lcp.start()
        writes.append(lcp)
        for c in range(max(0, N_CHUNKS - NBUF), N_CHUNKS):
            writes[c].wait()

    return gather_k(table, idx_pad)


BB = 64


def _tc_body(g_ref, f_ref, w_ref, b_ref, o_ref):
    o_ref[:, :, :TOKEN_DIM] = g_ref[:, :L, :]
    acc = lax.dot_general(
        f_ref[...],
        w_ref[...],
        (((2,), (1,)), ((), ())),
        preferred_element_type=jnp.float32,
    )
    o_ref[:, :, TOKEN_DIM:] = acc + b_ref[...]


def _tc_project_concat(gathered3d, features, W, b3d):
    return pl.pallas_call(
        _tc_body,
        grid=(B // BB,),
        in_specs=[
            pl.BlockSpec((BB, LP, TOKEN_DIM), lambda i: (i, 0, 0)),
            pl.BlockSpec((BB, L, FEAT_DIM), lambda i: (i, 0, 0)),
            pl.BlockSpec((FEAT_EMB_DIM, FEAT_DIM), lambda i: (0, 0)),
            pl.BlockSpec((1, 1, FEAT_EMB_DIM), lambda i: (0, 0, 0)),
        ],
        out_specs=pl.BlockSpec((BB, L, OUT_DIM), lambda i: (i, 0, 0)),
        out_shape=jax.ShapeDtypeStruct((B, L, OUT_DIM), jnp.float32),
    )(gathered3d, features, W, b3d)


@jax.jit
def kernel(tokens, features, table, W, b):
    pad_idx = (
        jnp.arange(B, dtype=jnp.int32)[:, None] * (LP - L)
        + jnp.arange(LP - L, dtype=jnp.int32)[None, :]
    )
    idx_pad = jnp.concatenate([tokens.astype(jnp.int32), pad_idx], axis=1)
    gathered = _sc_gather(table, idx_pad.reshape(N_PAD_ROWS))
    out = _tc_project_concat(
        gathered.reshape(B, LP, TOKEN_DIM),
        features,
        W,
        b.reshape(1, 1, FEAT_EMB_DIM),
    )
    return out
